# Initial kernel scaffold; baseline (speedup 1.0000x reference)
import jax
import jax.numpy as jnp
from jax.experimental import pallas as pl


def kernel(xs, xt, alpha):
    raise NotImplementedError("write your pallas kernel here")


# speedup vs baseline: 1.2888x; 1.2888x over previous
import functools

import numpy as np

import jax
import jax.numpy as jnp
from jax import lax
from jax.experimental import pallas as pl
from jax.experimental.pallas import tpu as pltpu
from jax.experimental.pallas import tpu_sc as plsc

ROWS = 1024
N = 4096
M = 2 * N
L = 16
NC = 2
NS = 16
NW = NC * NS
RPW = ROWS // NW
NV = M // L
CHUNK = M // L
RADIX = 256
HSIZE = RADIX * L

_I32_MIN = np.int32(-(2**31))


def _to_key(v):
    b = lax.bitcast_convert_type(v, jnp.int32)
    m = lax.shift_right_arithmetic(b, 31)
    return lax.bitwise_xor(b, lax.bitwise_or(m, _I32_MIN))


def _phys(p):
    return lax.bitwise_or(
        lax.shift_left(lax.bitwise_and(p, CHUNK - 1), 4),
        lax.shift_right_logical(p, 9))


def _digit(k, shift):
    return lax.bitwise_and(lax.shift_right_arithmetic(k, shift), RADIX - 1)


def _sc_body(xs_hbm, xt_hbm, out_hbm,
             raw_s, raw_t, key_a, key_b, side_a, side_b, hist, psum, accv):
    cid = lax.axis_index("c")
    sid = lax.axis_index("s")
    wid = cid * NS + sid
    lane = lax.iota(jnp.int32, L)
    ones = jnp.ones((L,), jnp.int32)
    zeros = jnp.zeros((L,), jnp.int32)

    def zero_hist(i, _):
        hist[pl.ds(i * L, L)] = zeros
        return 0

    def hist_pass(inkey, shift):
        def body(i, _):
            k = inkey[pl.ds(i * L, L)]
            idx = _digit(k, shift) * L + lane
            plsc.addupdate_scatter(hist, [idx], ones)
            return 0
        lax.fori_loop(0, NV, body, 0)

    def prefix_pass():
        def body_a(i, carry):
            psum[i] = carry
            return carry + jnp.sum(hist[pl.ds(i * L, L)])
        lax.fori_loop(0, RADIX, body_a, jnp.int32(0))

        def body_c(i, _):
            v = hist[pl.ds(i * L, L)]
            cs = plsc.cumsum(v)
            hist[pl.ds(i * L, L)] = cs - v + psum[i]
            return 0
        lax.fori_loop(0, RADIX, body_c, 0)

    def permute_pass(inkey, inside, outkey, outside, shift, first, last):
        def body(i, _):
            k = inkey[pl.ds(i * L, L)]
            if first:
                side = jnp.where(lane < (L // 2), ones, -ones)
            else:
                side = inside[pl.ds(i * L, L)]
            idx = _digit(k, shift) * L + lane
            off = plsc.load_gather(hist, [idx])
            dest = off if last else _phys(off)
            if not last:
                plsc.store_scatter(outkey, [dest], k)
            plsc.store_scatter(outside, [dest], side)
            plsc.store_scatter(hist, [idx], off + 1)
            return 0
        lax.fori_loop(0, NV, body, 0)

    def radix_pass(inkey, inside, outkey, outside, shift, first, last):
        lax.fori_loop(0, RADIX, zero_hist, 0)
        hist_pass(inkey, shift)
        prefix_pass()
        permute_pass(inkey, inside, outkey, outside, shift, first, last)

    def row_body(r, acc):
        row = wid * RPW + r
        pltpu.sync_copy(xs_hbm.at[row], raw_s)
        pltpu.sync_copy(xt_hbm.at[row], raw_t)

        def pre_s(u, _):
            v = raw_s[pl.ds(u * L, L)]
            p = u * L + lane
            plsc.store_scatter(key_a, [_phys(p)], _to_key(v))
            return 0
        def pre_t(u, _):
            v = raw_t[pl.ds(u * L, L)]
            p = N + u * L + lane
            plsc.store_scatter(key_a, [_phys(p)], _to_key(v))
            return 0
        lax.fori_loop(0, N // L, pre_s, 0)
        lax.fori_loop(0, N // L, pre_t, 0)

        radix_pass(key_a, None, key_b, side_b, 0, True, False)
        radix_pass(key_b, side_b, key_a, side_a, 8, False, False)
        radix_pass(key_a, side_a, key_b, side_b, 16, False, False)
        radix_pass(key_b, side_b, None, side_a, 24, False, True)

        def walk(i, carry):
            d0, mx, mn = carry
            s = side_a[pl.ds(i * L, L)]
            d = plsc.cumsum(s) + d0
            return (d0 + jnp.sum(s), jnp.maximum(mx, d), jnp.minimum(mn, d))
        d0, mx, mn = lax.fori_loop(
            0, NV, walk, (jnp.int32(0), zeros, zeros))
        dn = jnp.maximum(jnp.max(mx), -jnp.min(mn))

        f = dn.astype(jnp.float32)
        e = (f * f) * jnp.float32(-1.0 / N)
        val = jnp.float32(2.0) * jnp.exp(lax.broadcast(e, (L,)))
        return acc + jnp.where(lane < 1, val, jnp.float32(0.0))

    acc = lax.fori_loop(0, RPW, row_body, jnp.zeros((L,), jnp.float32))
    accv[...] = acc
    pltpu.sync_copy(accv, out_hbm.at[wid])


def kernel(xs, xt, alpha):
    del alpha
    mesh = plsc.VectorSubcoreMesh(
        core_axis_name="c", subcore_axis_name="s",
        num_cores=NC, num_subcores=NS)
    out = pl.kernel(
        _sc_body,
        out_type=jax.ShapeDtypeStruct((NW, L), jnp.float32),
        mesh=mesh,
        compiler_params=pltpu.CompilerParams(needs_layout_passes=False),
        scratch_types=[
            pltpu.VMEM((N,), jnp.float32),
            pltpu.VMEM((N,), jnp.float32),
            pltpu.VMEM((M,), jnp.int32),
            pltpu.VMEM((M,), jnp.int32),
            pltpu.VMEM((M,), jnp.int32),
            pltpu.VMEM((M,), jnp.int32),
            pltpu.VMEM((HSIZE,), jnp.int32),
            pltpu.SMEM((RADIX,), jnp.int32),
            pltpu.VMEM((L,), jnp.float32),
        ],
    )(xs, xt)
    return jnp.sum(out) / ROWS


# speedup vs baseline: 1.7161x; 1.3315x over previous
import functools

import numpy as np

import jax
import jax.numpy as jnp
from jax import lax
from jax.experimental import pallas as pl
from jax.experimental.pallas import tpu as pltpu
from jax.experimental.pallas import tpu_sc as plsc

ROWS = 1024
N = 4096
M = 2 * N
L = 16
NC = 2
NS = 16
NW = NC * NS
RPW = ROWS // NW
NV = M // L
CHUNK = M // L
RADIX = 256
HSIZE = RADIX * L

_I32_MIN = np.int32(-(2**31))


def _to_key(v):
    b = lax.bitcast_convert_type(v, jnp.int32)
    m = lax.shift_right_arithmetic(b, 31)
    return lax.bitwise_xor(b, lax.bitwise_or(m, _I32_MIN))


def _phys(p):
    return lax.bitwise_or(
        lax.shift_left(lax.bitwise_and(p, CHUNK - 1), 4),
        lax.shift_right_logical(p, 9))


def _digit(k, shift):
    return lax.bitwise_and(lax.shift_right_arithmetic(k, shift), RADIX - 1)


def _sc_body(xs_hbm, xt_hbm, out_hbm,
             raw_s, raw_t, key_a, key_b, side_a, side_b, hist, psum, accv):
    cid = lax.axis_index("c")
    sid = lax.axis_index("s")
    wid = cid * NS + sid
    lane = lax.iota(jnp.int32, L)
    ones = jnp.ones((L,), jnp.int32)
    zeros = jnp.zeros((L,), jnp.int32)

    def zero_hist(i, _):
        hist[pl.ds(i * L, L)] = zeros
        return 0

    def hist_pass(inkey, shift):
        def body(i, _):
            k = inkey[pl.ds(i * L, L)]
            idx = _digit(k, shift) * L + lane
            plsc.addupdate_scatter(hist, [idx], ones)
            return 0
        lax.fori_loop(0, NV, body, 0, unroll=8)

    def prefix_pass():
        def body_a(i, carry):
            psum[i] = carry
            return carry + jnp.sum(hist[pl.ds(i * L, L)])
        lax.fori_loop(0, RADIX, body_a, jnp.int32(0), unroll=4)

        def body_c(i, _):
            v = hist[pl.ds(i * L, L)]
            cs = plsc.cumsum(v)
            hist[pl.ds(i * L, L)] = cs - v + psum[i]
            return 0
        lax.fori_loop(0, RADIX, body_c, 0, unroll=8)

    def permute_pass(inkey, inside, outkey, outside, shift, first, last):
        def body(i, _):
            k = inkey[pl.ds(i * L, L)]
            if first:
                side = jnp.where(lane < (L // 2), ones, -ones)
            else:
                side = inside[pl.ds(i * L, L)]
            idx = _digit(k, shift) * L + lane
            off = plsc.load_gather(hist, [idx])
            dest = off if last else _phys(off)
            if not last:
                plsc.store_scatter(outkey, [dest], k)
            plsc.store_scatter(outside, [dest], side)
            plsc.store_scatter(hist, [idx], off + 1)
            return 0
        lax.fori_loop(0, NV, body, 0, unroll=4)

    def radix_pass(inkey, inside, outkey, outside, shift, first, last):
        lax.fori_loop(0, RADIX, zero_hist, 0, unroll=8)
        hist_pass(inkey, shift)
        prefix_pass()
        permute_pass(inkey, inside, outkey, outside, shift, first, last)

    def row_body(r, acc):
        row = wid * RPW + r
        pltpu.sync_copy(xs_hbm.at[row], raw_s)
        pltpu.sync_copy(xt_hbm.at[row], raw_t)

        def pre_s(u, _):
            v = raw_s[pl.ds(u * L, L)]
            p = u * L + lane
            plsc.store_scatter(key_a, [_phys(p)], _to_key(v))
            return 0
        def pre_t(u, _):
            v = raw_t[pl.ds(u * L, L)]
            p = N + u * L + lane
            plsc.store_scatter(key_a, [_phys(p)], _to_key(v))
            return 0
        lax.fori_loop(0, N // L, pre_s, 0, unroll=8)
        lax.fori_loop(0, N // L, pre_t, 0, unroll=8)

        radix_pass(key_a, None, key_b, side_b, 0, True, False)
        radix_pass(key_b, side_b, key_a, side_a, 8, False, False)
        radix_pass(key_a, side_a, key_b, side_b, 16, False, False)
        radix_pass(key_b, side_b, None, side_a, 24, False, True)

        def walk(i, carry):
            d0, mx, mn = carry
            s = side_a[pl.ds(i * L, L)]
            d = plsc.cumsum(s) + d0
            return (d0 + jnp.sum(s), jnp.maximum(mx, d), jnp.minimum(mn, d))
        d0, mx, mn = lax.fori_loop(
            0, NV, walk, (jnp.int32(0), zeros, zeros), unroll=4)
        dn = jnp.maximum(jnp.max(mx), -jnp.min(mn))

        f = dn.astype(jnp.float32)
        e = (f * f) * jnp.float32(-1.0 / N)
        val = jnp.float32(2.0) * jnp.exp(lax.broadcast(e, (L,)))
        return acc + jnp.where(lane < 1, val, jnp.float32(0.0))

    acc = lax.fori_loop(0, RPW, row_body, jnp.zeros((L,), jnp.float32))
    accv[...] = acc
    pltpu.sync_copy(accv, out_hbm.at[wid])


def kernel(xs, xt, alpha):
    del alpha
    mesh = plsc.VectorSubcoreMesh(
        core_axis_name="c", subcore_axis_name="s",
        num_cores=NC, num_subcores=NS)
    out = pl.kernel(
        _sc_body,
        out_type=jax.ShapeDtypeStruct((NW, L), jnp.float32),
        mesh=mesh,
        compiler_params=pltpu.CompilerParams(needs_layout_passes=False),
        scratch_types=[
            pltpu.VMEM((N,), jnp.float32),
            pltpu.VMEM((N,), jnp.float32),
            pltpu.VMEM((M,), jnp.int32),
            pltpu.VMEM((M,), jnp.int32),
            pltpu.VMEM((M,), jnp.int32),
            pltpu.VMEM((M,), jnp.int32),
            pltpu.VMEM((HSIZE,), jnp.int32),
            pltpu.SMEM((RADIX,), jnp.int32),
            pltpu.VMEM((L,), jnp.float32),
        ],
    )(xs, xt)
    return jnp.sum(out) / ROWS
